# baseline (speedup 1.0000x reference)
import functools

import jax
import jax.numpy as jnp
from jax import lax
from jax.experimental import pallas as pl
from jax.experimental.pallas import tpu as pltpu
from jax.experimental.pallas import tpu_sc as plsc

N = 10000
E = 320000
D = 128

ROWS = 1000
NB = N // ROWS

NC = 2
NS = 16
NW = NC * NS
EW = E // NW
CH = 128
NCHT = E // CH
T = NCHT // NW
TAIL = NCHT - T * NW
C = 100
NCHUNK = EW // C
SUB = 25
NSUP = NCHUNK // SUB
NPAD = 10240
SLAB = NPAD // NS

_MESH = plsc.VectorSubcoreMesh(core_axis_name="c", subcore_axis_name="s")


def _k1_body(f_ref, w1_ref, b1_ref, wga_ref, wgb_ref, d0_ref, d1_ref,
             nf_ref, xws_ref, dinv_ref):
    f = f_ref[...]
    h = jnp.dot(f, w1_ref[...], preferred_element_type=jnp.float32) + b1_ref[...]
    nf = jnp.where(h > 0, h, jnp.exp(h) - 1.0)
    nf_ref[...] = nf
    xw = (jnp.dot(nf, wga_ref[...], preferred_element_type=jnp.float32)
          + jnp.dot(f, wgb_ref[...], preferred_element_type=jnp.float32))
    deg = d0_ref[...] + d1_ref[...] + 1.0
    dinv = lax.rsqrt(deg)
    dinv_ref[...] = dinv
    xws_ref[...] = xw * dinv


_k1 = pl.pallas_call(
    _k1_body,
    grid=(NB,),
    in_specs=[
        pl.BlockSpec((ROWS, D), lambda i: (i, 0)),
        pl.BlockSpec((D, D), lambda i: (0, 0)),
        pl.BlockSpec((1, D), lambda i: (0, 0)),
        pl.BlockSpec((D, D), lambda i: (0, 0)),
        pl.BlockSpec((D, D), lambda i: (0, 0)),
        pl.BlockSpec((ROWS, 1), lambda i: (i, 0)),
        pl.BlockSpec((ROWS, 1), lambda i: (i, 0)),
    ],
    out_specs=[
        pl.BlockSpec((ROWS, D), lambda i: (i, 0)),
        pl.BlockSpec((ROWS, D), lambda i: (i, 0)),
        pl.BlockSpec((ROWS, 1), lambda i: (i, 0)),
    ],
    out_shape=[
        jax.ShapeDtypeStruct((N, D), jnp.float32),
        jax.ShapeDtypeStruct((N, D), jnp.float32),
        jax.ShapeDtypeStruct((N, 1), jnp.float32),
    ],
)


@functools.partial(
    pl.kernel,
    out_type=[jax.ShapeDtypeStruct((NPAD,), jnp.float32),
              jax.ShapeDtypeStruct((NPAD,), jnp.float32)],
    mesh=_MESH,
    scratch_types=[
        pltpu.VMEM((SUB, C), jnp.int32),
        pltpu.VMEM((112,), jnp.float32),
        pltpu.VMEM((SLAB,), jnp.float32),
        pltpu.VMEM_SHARED((NPAD,), jnp.float32),
    ],
)
def _sc_deg(dst4_hbm, out0_hbm, out1_hbm, didx_v, ones_v, zbuf_v, deg_sh):
    c = lax.axis_index("c")
    s = lax.axis_index("s")
    wid = c * NS + s

    def fill_ones(i, _):
        ones_v[pl.ds(i * 16, 16)] = jnp.full((16,), 1.0, jnp.float32)
        return 0

    lax.fori_loop(0, 112 // 16, fill_ones, 0)

    def fill_zero(i, _):
        zbuf_v[pl.ds(i * 16, 16)] = jnp.zeros((16,), jnp.float32)
        return 0

    lax.fori_loop(0, SLAB // 16, fill_zero, 0)

    pltpu.sync_copy(zbuf_v, deg_sh.at[pl.ds(s * SLAB, SLAB)])
    plsc.subcore_barrier()

    def sup(k, _):
        pltpu.sync_copy(dst4_hbm.at[wid, k], didx_v)

        def body(j, _):
            pltpu.sync_copy(ones_v.at[pl.ds(0, C)], deg_sh.at[didx_v.at[j]],
                            add=True)
            return 0

        lax.fori_loop(0, SUB, body, 0)
        return 0

    lax.fori_loop(0, NSUP, sup, 0)
    plsc.subcore_barrier()

    @pl.when(c == 0)
    def _():
        pltpu.sync_copy(deg_sh.at[pl.ds(s * SLAB, SLAB)],
                        out0_hbm.at[pl.ds(s * SLAB, SLAB)])

    @pl.when(c == 1)
    def _():
        pltpu.sync_copy(deg_sh.at[pl.ds(s * SLAB, SLAB)],
                        out1_hbm.at[pl.ds(s * SLAB, SLAB)])


@functools.partial(
    pl.kernel,
    out_type=[jax.ShapeDtypeStruct((NPAD, D), jnp.float32),
              jax.ShapeDtypeStruct((NPAD, D), jnp.float32)],
    mesh=_MESH,
    scratch_types=[
        pltpu.VMEM((SUB, C), jnp.int32),
        pltpu.VMEM((SUB, C), jnp.int32),
        pltpu.VMEM((3, C, D), jnp.float32),
        pltpu.VMEM_SHARED((NPAD, D), jnp.float32),
    ] + [pltpu.SemaphoreType.DMA] * 3,
)
def _sc_agg(e5_hbm, xws_hbm, out0_hbm, out1_hbm,
            sidx_v, didx_v, rows_v, agg_sh, *sems):
    c = lax.axis_index("c")
    s = lax.axis_index("s")
    wid = c * NS + s

    def fill_zero(i, _):
        rows_v[0, i // 8, pl.ds((i % 8) * 16, 16)] = jnp.zeros((16,),
                                                               jnp.float32)
        return 0

    lax.fori_loop(0, 80 * 8, fill_zero, 0)

    def zinit(i, _):
        pltpu.sync_copy(rows_v.at[0, pl.ds(0, 80)],
                        agg_sh.at[pl.ds(s * SLAB + i * 80, 80)])
        return 0

    lax.fori_loop(0, SLAB // 80, zinit, 0)
    plsc.subcore_barrier()

    def gather(j, buf):
        return pltpu.make_async_copy(xws_hbm.at[sidx_v.at[j]],
                                     rows_v.at[buf], sems[buf])

    def scat(j, buf):
        pltpu.sync_copy(rows_v.at[buf], agg_sh.at[didx_v.at[j]], add=True)

    def sup(k, _):
        pltpu.sync_copy(e5_hbm.at[0, wid, k], sidx_v)
        pltpu.sync_copy(e5_hbm.at[1, wid, k], didx_v)

        gather(0, 0).start()
        gather(1, 1).start()
        for j in range(SUB):
            if j + 2 < SUB:
                gather(j + 2, (j + 2) % 3).start()
            gather(j, j % 3).wait()
            scat(j, j % 3)
        return 0

    lax.fori_loop(0, NSUP, sup, 0)
    plsc.subcore_barrier()

    @pl.when(c == 0)
    def _():
        pltpu.sync_copy(agg_sh.at[pl.ds(s * SLAB, SLAB)],
                        out0_hbm.at[pl.ds(s * SLAB, SLAB)])

    @pl.when(c == 1)
    def _():
        pltpu.sync_copy(agg_sh.at[pl.ds(s * SLAB, SLAB)],
                        out1_hbm.at[pl.ds(s * SLAB, SLAB)])


def _k3_body(nf_ref, f_ref, a0_ref, a1_ref, xws_ref, dinv_ref,
             wca_ref, wcb_ref, bc_ref, bgc_ref, out_ref):
    dinv = dinv_ref[...]
    gc = (a0_ref[...] + a1_ref[...] + xws_ref[...]) * dinv + bgc_ref[...]
    h = (jnp.dot(nf_ref[...], wca_ref[...], preferred_element_type=jnp.float32)
         + jnp.dot(gc, wcb_ref[...], preferred_element_type=jnp.float32)
         + bc_ref[...])
    out_ref[:, :D] = jnp.where(h > 0, h, jnp.exp(h) - 1.0)
    out_ref[:, D:] = f_ref[...]


_k3 = pl.pallas_call(
    _k3_body,
    grid=(NB,),
    in_specs=[
        pl.BlockSpec((ROWS, D), lambda i: (i, 0)),
        pl.BlockSpec((ROWS, D), lambda i: (i, 0)),
        pl.BlockSpec((ROWS, D), lambda i: (i, 0)),
        pl.BlockSpec((ROWS, D), lambda i: (i, 0)),
        pl.BlockSpec((ROWS, D), lambda i: (i, 0)),
        pl.BlockSpec((ROWS, 1), lambda i: (i, 0)),
        pl.BlockSpec((D, D), lambda i: (0, 0)),
        pl.BlockSpec((D, D), lambda i: (0, 0)),
        pl.BlockSpec((1, D), lambda i: (0, 0)),
        pl.BlockSpec((1, D), lambda i: (0, 0)),
    ],
    out_specs=pl.BlockSpec((ROWS, 2 * D), lambda i: (i, 0)),
    out_shape=jax.ShapeDtypeStruct((N, 2 * D), jnp.float32),
)


def kernel(feats, edges, batch, W1, b1, Wgc, bgc, Wc, bc):
    dst4 = edges[1].reshape(NW, NSUP, SUB, C)
    deg0, deg1 = _sc_deg(dst4)

    nfeats, xws, dinv = _k1(feats, W1, b1.reshape(1, D), Wgc[:D], Wgc[D:],
                            deg0.reshape(NPAD, 1), deg1.reshape(NPAD, 1))

    e5 = edges.reshape(2, NW, NSUP, SUB, C)
    agg0, agg1 = _sc_agg(e5, xws)

    out_feats = _k3(nfeats, feats, agg0, agg1, xws, dinv,
                    Wc[:D], Wc[D:], bc.reshape(1, D), bgc.reshape(1, D))
    return (out_feats, edges, batch)


# speedup vs baseline: 1.0570x; 1.0570x over previous
import functools

import jax
import jax.numpy as jnp
from jax import lax
from jax.experimental import pallas as pl
from jax.experimental.pallas import tpu as pltpu
from jax.experimental.pallas import tpu_sc as plsc

N = 10000
E = 320000
D = 128

ROWS = 1000
NB = N // ROWS

NC = 2
NS = 16
NW = NC * NS
EW = E // NW
CH = 128
NCHT = E // CH
T = NCHT // NW
TAIL = NCHT - T * NW
C = 100
NCHUNK = EW // C
SUB = 25
NSUP = NCHUNK // SUB
NPAD = 10240
SLAB = NPAD // NS

_MESH = plsc.VectorSubcoreMesh(core_axis_name="c", subcore_axis_name="s")


def _k1_body(f_ref, w1_ref, b1_ref, wga_ref, wgb_ref, d0_ref, d1_ref,
             nf_ref, xws_ref, dinv_ref):
    f = f_ref[...]
    h = jnp.dot(f, w1_ref[...], preferred_element_type=jnp.float32) + b1_ref[...]
    nf = jnp.where(h > 0, h, jnp.exp(h) - 1.0)
    nf_ref[...] = nf
    xw = (jnp.dot(nf, wga_ref[...], preferred_element_type=jnp.float32)
          + jnp.dot(f, wgb_ref[...], preferred_element_type=jnp.float32))
    deg = d0_ref[...] + d1_ref[...] + 1.0
    dinv = lax.rsqrt(deg)
    dinv_ref[...] = dinv
    xws_ref[...] = xw * dinv


_k1 = pl.pallas_call(
    _k1_body,
    grid=(NB,),
    in_specs=[
        pl.BlockSpec((ROWS, D), lambda i: (i, 0)),
        pl.BlockSpec((D, D), lambda i: (0, 0)),
        pl.BlockSpec((1, D), lambda i: (0, 0)),
        pl.BlockSpec((D, D), lambda i: (0, 0)),
        pl.BlockSpec((D, D), lambda i: (0, 0)),
        pl.BlockSpec((ROWS, 1), lambda i: (i, 0)),
        pl.BlockSpec((ROWS, 1), lambda i: (i, 0)),
    ],
    out_specs=[
        pl.BlockSpec((ROWS, D), lambda i: (i, 0)),
        pl.BlockSpec((ROWS, D), lambda i: (i, 0)),
        pl.BlockSpec((ROWS, 1), lambda i: (i, 0)),
    ],
    out_shape=[
        jax.ShapeDtypeStruct((N, D), jnp.float32),
        jax.ShapeDtypeStruct((N, D), jnp.float32),
        jax.ShapeDtypeStruct((N, 1), jnp.float32),
    ],
)


@functools.partial(
    pl.kernel,
    out_type=[jax.ShapeDtypeStruct((NPAD,), jnp.float32),
              jax.ShapeDtypeStruct((NPAD,), jnp.float32)],
    mesh=_MESH,
    scratch_types=[
        pltpu.VMEM((SUB, C), jnp.int32),
        pltpu.VMEM((112,), jnp.float32),
        pltpu.VMEM((SLAB,), jnp.float32),
        pltpu.VMEM_SHARED((NPAD,), jnp.float32),
    ],
)
def _sc_deg(e5_hbm, out0_hbm, out1_hbm, didx_v, ones_v, zbuf_v, deg_sh):
    c = lax.axis_index("c")
    s = lax.axis_index("s")
    wid = c * NS + s

    def fill_ones(i, _):
        ones_v[pl.ds(i * 16, 16)] = jnp.full((16,), 1.0, jnp.float32)
        return 0

    lax.fori_loop(0, 112 // 16, fill_ones, 0)

    def fill_zero(i, _):
        zbuf_v[pl.ds(i * 16, 16)] = jnp.zeros((16,), jnp.float32)
        return 0

    lax.fori_loop(0, SLAB // 16, fill_zero, 0)

    pltpu.sync_copy(zbuf_v, deg_sh.at[pl.ds(s * SLAB, SLAB)])
    plsc.subcore_barrier()

    def sup(k, _):
        pltpu.sync_copy(e5_hbm.at[1, wid, k], didx_v)

        def body(j, _):
            pltpu.sync_copy(ones_v.at[pl.ds(0, C)], deg_sh.at[didx_v.at[j]],
                            add=True)
            return 0

        lax.fori_loop(0, SUB, body, 0)
        return 0

    lax.fori_loop(0, NSUP, sup, 0)
    plsc.subcore_barrier()

    @pl.when(c == 0)
    def _():
        pltpu.sync_copy(deg_sh.at[pl.ds(s * SLAB, SLAB)],
                        out0_hbm.at[pl.ds(s * SLAB, SLAB)])

    @pl.when(c == 1)
    def _():
        pltpu.sync_copy(deg_sh.at[pl.ds(s * SLAB, SLAB)],
                        out1_hbm.at[pl.ds(s * SLAB, SLAB)])


@functools.partial(
    pl.kernel,
    out_type=[jax.ShapeDtypeStruct((NPAD, D), jnp.float32),
              jax.ShapeDtypeStruct((NPAD, D), jnp.float32)],
    mesh=_MESH,
    scratch_types=[
        pltpu.VMEM((SUB, C), jnp.int32),
        pltpu.VMEM((SUB, C), jnp.int32),
        pltpu.VMEM((3, C, D), jnp.float32),
        pltpu.VMEM_SHARED((NPAD, D), jnp.float32),
    ] + [pltpu.SemaphoreType.DMA] * 3,
)
def _sc_agg(e5_hbm, xws_hbm, out0_hbm, out1_hbm,
            sidx_v, didx_v, rows_v, agg_sh, *sems):
    c = lax.axis_index("c")
    s = lax.axis_index("s")
    wid = c * NS + s

    def fill_zero(i, _):
        rows_v[0, i // 8, pl.ds((i % 8) * 16, 16)] = jnp.zeros((16,),
                                                               jnp.float32)
        return 0

    lax.fori_loop(0, 80 * 8, fill_zero, 0)

    def zinit(i, _):
        pltpu.sync_copy(rows_v.at[0, pl.ds(0, 80)],
                        agg_sh.at[pl.ds(s * SLAB + i * 80, 80)])
        return 0

    lax.fori_loop(0, SLAB // 80, zinit, 0)
    plsc.subcore_barrier()

    def gather(j, buf):
        return pltpu.make_async_copy(xws_hbm.at[sidx_v.at[j]],
                                     rows_v.at[buf], sems[buf])

    def scat(j, buf):
        pltpu.sync_copy(rows_v.at[buf], agg_sh.at[didx_v.at[j]], add=True)

    def sup(k, _):
        pltpu.sync_copy(e5_hbm.at[0, wid, k], sidx_v)
        pltpu.sync_copy(e5_hbm.at[1, wid, k], didx_v)

        gather(0, 0).start()
        gather(1, 1).start()
        for j in range(SUB):
            if j + 2 < SUB:
                gather(j + 2, (j + 2) % 3).start()
            gather(j, j % 3).wait()
            scat(j, j % 3)
        return 0

    lax.fori_loop(0, NSUP, sup, 0)
    plsc.subcore_barrier()

    @pl.when(c == 0)
    def _():
        pltpu.sync_copy(agg_sh.at[pl.ds(s * SLAB, SLAB)],
                        out0_hbm.at[pl.ds(s * SLAB, SLAB)])

    @pl.when(c == 1)
    def _():
        pltpu.sync_copy(agg_sh.at[pl.ds(s * SLAB, SLAB)],
                        out1_hbm.at[pl.ds(s * SLAB, SLAB)])


def _k3_body(nf_ref, f_ref, a0_ref, a1_ref, xws_ref, dinv_ref,
             wca_ref, wcb_ref, bc_ref, bgc_ref, out_ref):
    dinv = dinv_ref[...]
    gc = (a0_ref[...] + a1_ref[...] + xws_ref[...]) * dinv + bgc_ref[...]
    h = (jnp.dot(nf_ref[...], wca_ref[...], preferred_element_type=jnp.float32)
         + jnp.dot(gc, wcb_ref[...], preferred_element_type=jnp.float32)
         + bc_ref[...])
    out_ref[:, :D] = jnp.where(h > 0, h, jnp.exp(h) - 1.0)
    out_ref[:, D:] = f_ref[...]


_k3 = pl.pallas_call(
    _k3_body,
    grid=(NB,),
    in_specs=[
        pl.BlockSpec((ROWS, D), lambda i: (i, 0)),
        pl.BlockSpec((ROWS, D), lambda i: (i, 0)),
        pl.BlockSpec((ROWS, D), lambda i: (i, 0)),
        pl.BlockSpec((ROWS, D), lambda i: (i, 0)),
        pl.BlockSpec((ROWS, D), lambda i: (i, 0)),
        pl.BlockSpec((ROWS, 1), lambda i: (i, 0)),
        pl.BlockSpec((D, D), lambda i: (0, 0)),
        pl.BlockSpec((D, D), lambda i: (0, 0)),
        pl.BlockSpec((1, D), lambda i: (0, 0)),
        pl.BlockSpec((1, D), lambda i: (0, 0)),
    ],
    out_specs=pl.BlockSpec((ROWS, 2 * D), lambda i: (i, 0)),
    out_shape=jax.ShapeDtypeStruct((N, 2 * D), jnp.float32),
)


def kernel(feats, edges, batch, W1, b1, Wgc, bgc, Wc, bc):
    e5 = edges.reshape(2, NW, NSUP, SUB, C)
    deg0, deg1 = _sc_deg(e5)

    nfeats, xws, dinv = _k1(feats, W1, b1.reshape(1, D), Wgc[:D], Wgc[D:],
                            deg0.reshape(NPAD, 1), deg1.reshape(NPAD, 1))

    agg0, agg1 = _sc_agg(e5, xws)

    out_feats = _k3(nfeats, feats, agg0, agg1, xws, dinv,
                    Wc[:D], Wc[D:], bc.reshape(1, D), bgc.reshape(1, D))
    return (out_feats, edges, batch)
